# baseline (speedup 1.0000x reference)
import functools
import math

import jax
import jax.numpy as jnp
from jax import lax
from jax.experimental import pallas as pl
from jax.experimental.pallas import tpu as pltpu
from jax.experimental.pallas import tpu_sc as plsc

N = 10000
F = 128
N_PAD = 10240
BLK = 2048
NC, NS = 2, 16
NW = NC * NS
ROWS_PER_TILE = N_PAD // NS
E = 320000
CHUNK = 128
NUM_CHUNKS = 80
EDGES_PER_TILE = NUM_CHUNKS * CHUNK
E_PAD = NW * EDGES_PER_TILE
C_BN = 1.0 / math.sqrt(1.0 + 1e-5)


def _sc_body(compute_deg, p_hbm, src_hbm, dst_hbm, zero_hbm, *rest):
    if compute_deg:
        (zero1_hbm, acc_out, deg_out, src_v, dst_v, rows,
         ones_v, acc_sh, deg_sh, sem) = rest
    else:
        (acc_out, src_v, dst_v, rows, acc_sh, sem) = rest
    cid = lax.axis_index("c")
    sid = lax.axis_index("s")
    wid = sid * NC + cid
    r0 = sid * ROWS_PER_TILE

    pltpu.sync_copy(zero_hbm.at[pl.ds(r0, ROWS_PER_TILE)],
                    acc_sh.at[pl.ds(r0, ROWS_PER_TILE)])
    if compute_deg:
        @pl.loop(0, CHUNK // 16)
        def _fill_ones(i):
            ones_v[pl.ds(i * 16, 16)] = jnp.ones((16,), jnp.float32)
        pltpu.sync_copy(zero1_hbm.at[pl.ds(r0, ROWS_PER_TILE)],
                        deg_sh.at[pl.ds(r0, ROWS_PER_TILE)])
    plsc.subcore_barrier()

    ebase = wid * EDGES_PER_TILE

    @pl.loop(0, NUM_CHUNKS)
    def _chunk(i):
        off = ebase + i * CHUNK
        pltpu.sync_copy(src_hbm.at[pl.ds(off, CHUNK)], src_v)
        pltpu.sync_copy(dst_hbm.at[pl.ds(off, CHUNK)], dst_v)
        pltpu.async_copy(p_hbm.at[src_v], rows, sem).wait()
        pltpu.sync_copy(rows, acc_sh.at[dst_v], add=True)
        if compute_deg:
            pltpu.sync_copy(ones_v, deg_sh.at[dst_v], add=True)

    plsc.subcore_barrier()
    pltpu.sync_copy(acc_sh.at[pl.ds(r0, ROWS_PER_TILE)],
                    acc_out.at[cid, pl.ds(r0, ROWS_PER_TILE)])
    if compute_deg:
        pltpu.sync_copy(deg_sh.at[pl.ds(r0, ROWS_PER_TILE)],
                        deg_out.at[cid, pl.ds(r0, ROWS_PER_TILE)])


def _make_sc_kernel(compute_deg):
    out_type = [jax.ShapeDtypeStruct((NC, N_PAD, F), jnp.float32)]
    scratch = [
        pltpu.VMEM((CHUNK,), jnp.int32),
        pltpu.VMEM((CHUNK,), jnp.int32),
        pltpu.VMEM((CHUNK, F), jnp.float32),
    ]
    if compute_deg:
        out_type.append(jax.ShapeDtypeStruct((NC, N_PAD), jnp.float32))
        scratch.append(pltpu.VMEM((CHUNK,), jnp.float32))
    scratch.append(pltpu.VMEM_SHARED((N_PAD, F), jnp.float32))
    if compute_deg:
        scratch.append(pltpu.VMEM_SHARED((N_PAD,), jnp.float32))
    scratch.append(pltpu.SemaphoreType.DMA)
    return pl.kernel(
        functools.partial(_sc_body, compute_deg),
        out_type=tuple(out_type),
        mesh=plsc.VectorSubcoreMesh(core_axis_name="c", subcore_axis_name="s"),
        scratch_types=tuple(scratch),
    )


def _mm_in_body(x_ref, w_ref, b_ref, p_ref, r_ref):
    o = lax.dot_general(x_ref[...], w_ref[...], (((1,), (1,)), ((), ())),
                        preferred_element_type=jnp.float32)
    p_ref[...] = o[:, :F]
    r_ref[...] = o[:, F:] + b_ref[...]


def _fuse_body(parts_ref, degp_ref, r_ref, g_ref, be_ref, w_ref, b_ref,
               p_ref, rout_ref):
    acc = parts_ref[0] + parts_ref[1]
    deg = jnp.maximum(jnp.sum(degp_ref[...], axis=0), 1.0)
    h = acc / deg[:, None] + r_ref[...]
    h = jnp.maximum(g_ref[...] * (h * C_BN) + be_ref[...], 0.0)
    o = lax.dot_general(h, w_ref[...], (((1,), (1,)), ((), ())),
                        preferred_element_type=jnp.float32)
    p_ref[...] = o[:, :F]
    rout_ref[...] = o[:, F:] + b_ref[...]


def _head_body(parts_ref, degp_ref, r_ref, g_ref, be_ref, w_ref, b_ref,
               out_ref):
    acc = parts_ref[0] + parts_ref[1]
    deg = jnp.maximum(jnp.sum(degp_ref[...], axis=0), 1.0)
    h = acc / deg[:, None] + r_ref[...]
    h = jnp.maximum(g_ref[...] * (h * C_BN) + be_ref[...], 0.0)
    out_ref[...] = lax.dot_general(h, w_ref[...], (((1,), (1,)), ((), ())),
                                   preferred_element_type=jnp.float32) + b_ref[...]


_GRID = N_PAD // BLK
_row = pl.BlockSpec((BLK, F), lambda i: (i, 0))
_full2 = lambda a, b: pl.BlockSpec((a, b), lambda i: (0, 0))
_parts_spec = pl.BlockSpec((NC, BLK, F), lambda i: (0, i, 0))
_degp_spec = pl.BlockSpec((NC, BLK), lambda i: (0, i))

_mm_in = pl.pallas_call(
    _mm_in_body,
    grid=(_GRID,),
    in_specs=[_row, _full2(2 * F, F), _full2(1, F)],
    out_specs=[_row, _row],
    out_shape=[jax.ShapeDtypeStruct((N_PAD, F), jnp.float32)] * 2,
)

_fuse = pl.pallas_call(
    _fuse_body,
    grid=(_GRID,),
    in_specs=[_parts_spec, _degp_spec, _row, _full2(1, F), _full2(1, F),
              _full2(2 * F, F), _full2(1, F)],
    out_specs=[_row, _row],
    out_shape=[jax.ShapeDtypeStruct((N_PAD, F), jnp.float32)] * 2,
)

_head = pl.pallas_call(
    _head_body,
    grid=(_GRID,),
    in_specs=[_parts_spec, _degp_spec, _row, _full2(1, F), _full2(1, F),
              _full2(F, F), _full2(1, F)],
    out_specs=_row,
    out_shape=jax.ShapeDtypeStruct((N_PAD, F), jnp.float32),
)


def kernel(x, ei, W1l, b1l, W1r, W2l, b2l, W2r, W3l, b3l, W3r,
           g1, be1, g2, be2, g3, be3, Wh, bh):
    x_pad = jnp.pad(x, ((0, N_PAD - N), (0, 0)))
    src = jnp.concatenate([ei[0], jnp.zeros((E_PAD - E,), jnp.int32)])
    dst = jnp.concatenate([ei[1], jnp.full((E_PAD - E,), N, jnp.int32)])
    zeros = jnp.zeros((N_PAD, F), jnp.float32)
    zeros1 = jnp.zeros((N_PAD,), jnp.float32)

    row = lambda v: v.reshape(1, F)
    Wc1 = jnp.concatenate([W1l, W1r], axis=0)
    Wc2 = jnp.concatenate([W2l, W2r], axis=0)
    Wc3 = jnp.concatenate([W3l, W3r], axis=0)

    agg_deg = _make_sc_kernel(True)
    agg = _make_sc_kernel(False)

    p1, r1 = _mm_in(x_pad, Wc1, row(b1l))
    parts1, degp = agg_deg(p1, src, dst, zeros, zeros1)
    p2, r2 = _fuse(parts1, degp, r1, row(g1), row(be1), Wc2, row(b2l))
    (parts2,) = agg(p2, src, dst, zeros)
    p3, r3 = _fuse(parts2, degp, r2, row(g2), row(be2), Wc3, row(b3l))
    (parts3,) = agg(p3, src, dst, zeros)
    out = _head(parts3, degp, r3, row(g3), row(be3), Wh, row(bh))
    return out[:N]


# speedup vs baseline: 2.0589x; 2.0589x over previous
import functools
import math

import jax
import jax.numpy as jnp
from jax import lax
from jax.experimental import pallas as pl
from jax.experimental.pallas import tpu as pltpu
from jax.experimental.pallas import tpu_sc as plsc

N = 10000
F = 128
N_PAD = 10240
BLK = 2048
NC, NS = 2, 16
NW = NC * NS
ROWS_PER_TILE = N_PAD // NS
E = 320000
CHUNK = 128
NUM_CHUNKS = 80
EDGES_PER_TILE = NUM_CHUNKS * CHUNK
E_PAD = NW * EDGES_PER_TILE
C_BN = 1.0 / math.sqrt(1.0 + 1e-5)


def _sc_body(compute_deg, p_hbm, src_hbm, dst_hbm, zero_hbm, *rest):
    if compute_deg:
        (zero1_hbm, acc_out, deg_out, src_v, dst_v, rows,
         ones_v, acc_sh, deg_sh, sem) = rest
    else:
        (acc_out, src_v, dst_v, rows, acc_sh, sem) = rest
    cid = lax.axis_index("c")
    sid = lax.axis_index("s")
    wid = sid * NC + cid
    r0 = sid * ROWS_PER_TILE

    pltpu.sync_copy(zero_hbm.at[pl.ds(r0, ROWS_PER_TILE)],
                    acc_sh.at[pl.ds(r0, ROWS_PER_TILE)])
    if compute_deg:
        @pl.loop(0, CHUNK // 16)
        def _fill_ones(i):
            ones_v[pl.ds(i * 16, 16)] = jnp.ones((16,), jnp.float32)
        pltpu.sync_copy(zero1_hbm.at[pl.ds(r0, ROWS_PER_TILE)],
                        deg_sh.at[pl.ds(r0, ROWS_PER_TILE)])
    plsc.subcore_barrier()

    ebase = wid * EDGES_PER_TILE

    @pl.loop(0, NUM_CHUNKS)
    def _chunk(i):
        off = ebase + i * CHUNK
        pltpu.sync_copy(src_hbm.at[pl.ds(off, CHUNK)], src_v)
        pltpu.sync_copy(dst_hbm.at[pl.ds(off, CHUNK)], dst_v)
        pltpu.async_copy(p_hbm.at[src_v], rows, sem).wait()
        pltpu.sync_copy(rows, acc_sh.at[dst_v], add=True)
        if compute_deg:
            pltpu.sync_copy(ones_v, deg_sh.at[dst_v], add=True)

    plsc.subcore_barrier()
    pltpu.sync_copy(acc_sh.at[pl.ds(r0, ROWS_PER_TILE)],
                    acc_out.at[cid, pl.ds(r0, ROWS_PER_TILE)])
    if compute_deg:
        pltpu.sync_copy(deg_sh.at[pl.ds(r0, ROWS_PER_TILE)],
                        deg_out.at[cid, pl.ds(r0, ROWS_PER_TILE)])


def _make_sc_kernel(compute_deg):
    out_type = [jax.ShapeDtypeStruct((NC, N_PAD, F), jnp.float32)]
    scratch = [
        pltpu.VMEM((CHUNK,), jnp.int32),
        pltpu.VMEM((CHUNK,), jnp.int32),
        pltpu.VMEM((CHUNK, F), jnp.float32),
    ]
    if compute_deg:
        out_type.append(jax.ShapeDtypeStruct((NC, N_PAD), jnp.float32))
        scratch.append(pltpu.VMEM((CHUNK,), jnp.float32))
    scratch.append(pltpu.VMEM_SHARED((N_PAD, F), jnp.float32))
    if compute_deg:
        scratch.append(pltpu.VMEM_SHARED((N_PAD,), jnp.float32))
    scratch.append(pltpu.SemaphoreType.DMA)
    return pl.kernel(
        functools.partial(_sc_body, compute_deg),
        out_type=tuple(out_type),
        mesh=plsc.VectorSubcoreMesh(core_axis_name="c", subcore_axis_name="s"),
        scratch_types=tuple(scratch),
    )


def _mm_in_body(x_ref, w_ref, b_ref, p_ref, r_ref):
    o = lax.dot_general(x_ref[...], w_ref[...], (((1,), (1,)), ((), ())),
                        preferred_element_type=jnp.float32)
    p_ref[...] = o[:, :F]
    r_ref[...] = o[:, F:] + b_ref[...]


def _fuse_body(parts_ref, degp_ref, r_ref, g_ref, be_ref, w_ref, b_ref,
               p_ref, rout_ref):
    acc = parts_ref[0] + parts_ref[1]
    deg = jnp.maximum(jnp.sum(degp_ref[...], axis=0), 1.0)
    h = acc / deg[:, None] + r_ref[...]
    h = jnp.maximum(g_ref[...] * (h * C_BN) + be_ref[...], 0.0)
    o = lax.dot_general(h, w_ref[...], (((1,), (1,)), ((), ())),
                        preferred_element_type=jnp.float32)
    p_ref[...] = o[:, :F]
    rout_ref[...] = o[:, F:] + b_ref[...]


def _head_body(parts_ref, degp_ref, r_ref, g_ref, be_ref, w_ref, b_ref,
               out_ref):
    acc = parts_ref[0] + parts_ref[1]
    deg = jnp.maximum(jnp.sum(degp_ref[...], axis=0), 1.0)
    h = acc / deg[:, None] + r_ref[...]
    h = jnp.maximum(g_ref[...] * (h * C_BN) + be_ref[...], 0.0)
    out_ref[...] = lax.dot_general(h, w_ref[...], (((1,), (1,)), ((), ())),
                                   preferred_element_type=jnp.float32) + b_ref[...]


_GRID = N_PAD // BLK
_row = pl.BlockSpec((BLK, F), lambda i: (i, 0))
_full2 = lambda a, b: pl.BlockSpec((a, b), lambda i: (0, 0))
_parts_spec = pl.BlockSpec((NC, BLK, F), lambda i: (0, i, 0))
_degp_spec = pl.BlockSpec((NC, BLK), lambda i: (0, i))

_mm_in = pl.pallas_call(
    _mm_in_body,
    grid=(_GRID,),
    in_specs=[_row, _full2(2 * F, F), _full2(1, F)],
    out_specs=[_row, _row],
    out_shape=[jax.ShapeDtypeStruct((N_PAD, F), jnp.float32)] * 2,
)

_fuse = pl.pallas_call(
    _fuse_body,
    grid=(_GRID,),
    in_specs=[_parts_spec, _degp_spec, _row, _full2(1, F), _full2(1, F),
              _full2(2 * F, F), _full2(1, F)],
    out_specs=[_row, _row],
    out_shape=[jax.ShapeDtypeStruct((N_PAD, F), jnp.float32)] * 2,
)

_head = pl.pallas_call(
    _head_body,
    grid=(_GRID,),
    in_specs=[_parts_spec, _degp_spec, _row, _full2(1, F), _full2(1, F),
              _full2(F, F), _full2(1, F)],
    out_specs=_row,
    out_shape=jax.ShapeDtypeStruct((N_PAD, F), jnp.float32),
)


def kernel(x, ei, W1l, b1l, W1r, W2l, b2l, W2r, W3l, b3l, W3r,
           g1, be1, g2, be2, g3, be3, Wh, bh):
    x_pad = jnp.pad(x, ((0, N_PAD - N), (0, 0)))
    pad_i = jnp.arange(E_PAD - E, dtype=jnp.int32)
    src = jnp.concatenate([ei[0], pad_i * 13 % N])
    dst = jnp.concatenate([ei[1], N + pad_i % (N_PAD - N)])
    zeros = jnp.zeros((N_PAD, F), jnp.float32)
    zeros1 = jnp.zeros((N_PAD,), jnp.float32)

    row = lambda v: v.reshape(1, F)
    Wc1 = jnp.concatenate([W1l, W1r], axis=0)
    Wc2 = jnp.concatenate([W2l, W2r], axis=0)
    Wc3 = jnp.concatenate([W3l, W3r], axis=0)

    agg_deg = _make_sc_kernel(True)
    agg = _make_sc_kernel(False)

    p1, r1 = _mm_in(x_pad, Wc1, row(b1l))
    parts1, degp = agg_deg(p1, src, dst, zeros, zeros1)
    p2, r2 = _fuse(parts1, degp, r1, row(g1), row(be1), Wc2, row(b2l))
    (parts2,) = agg(p2, src, dst, zeros)
    p3, r3 = _fuse(parts2, degp, r2, row(g2), row(be2), Wc3, row(b3l))
    (parts3,) = agg(p3, src, dst, zeros)
    out = _head(parts3, degp, r3, row(g3), row(be3), Wh, row(bh))
    return out[:N]


# speedup vs baseline: 3.1984x; 1.5534x over previous
import functools
import math

import jax
import jax.numpy as jnp
from jax import lax
from jax.experimental import pallas as pl
from jax.experimental.pallas import tpu as pltpu
from jax.experimental.pallas import tpu_sc as plsc

N = 10000
F = 128
N_PAD = 10240
BLK = 2048
NC, NS = 2, 16
NW = NC * NS
ROWS_PER_TILE = N_PAD // NS
E = 320000
CHUNK = 128
NUM_CHUNKS = 80
EDGES_PER_TILE = NUM_CHUNKS * CHUNK
E_PAD = NW * EDGES_PER_TILE
C_BN = 1.0 / math.sqrt(1.0 + 1e-5)


def _sc_body(compute_deg, p_hbm, src_hbm, dst_hbm, zero_hbm, *rest):
    if compute_deg:
        (zero1_hbm, acc_out, deg_out, src_v0, dst_v0, src_v1, dst_v1,
         rows0, rows1, ones_v, acc_sh, deg_sh, sem0, sem1) = rest
    else:
        (acc_out, src_v0, dst_v0, src_v1, dst_v1, rows0, rows1,
         acc_sh, sem0, sem1) = rest
    cid = lax.axis_index("c")
    sid = lax.axis_index("s")
    wid = sid * NC + cid
    r0 = sid * ROWS_PER_TILE

    pltpu.sync_copy(zero_hbm.at[pl.ds(r0, ROWS_PER_TILE)],
                    acc_sh.at[pl.ds(r0, ROWS_PER_TILE)])
    if compute_deg:
        @pl.loop(0, CHUNK // 16)
        def _fill_ones(i):
            ones_v[pl.ds(i * 16, 16)] = jnp.ones((16,), jnp.float32)
        pltpu.sync_copy(zero1_hbm.at[pl.ds(r0, ROWS_PER_TILE)],
                        deg_sh.at[pl.ds(r0, ROWS_PER_TILE)])
    plsc.subcore_barrier()

    ebase = wid * EDGES_PER_TILE

    def load_idx(c, sv, dv):
        off = ebase + c * CHUNK
        pltpu.sync_copy(src_hbm.at[pl.ds(off, CHUNK)], sv)
        pltpu.sync_copy(dst_hbm.at[pl.ds(off, CHUNK)], dv)

    load_idx(0, src_v0, dst_v0)
    pltpu.async_copy(p_hbm.at[src_v0], rows0, sem0)

    @pl.loop(0, NUM_CHUNKS // 2)
    def _chunk(j):
        c0 = 2 * j
        load_idx(c0 + 1, src_v1, dst_v1)
        pltpu.async_copy(p_hbm.at[src_v1], rows1, sem1)
        pltpu.make_async_copy(p_hbm.at[src_v0], rows0, sem0).wait()
        pltpu.sync_copy(rows0, acc_sh.at[dst_v0], add=True)
        if compute_deg:
            pltpu.sync_copy(ones_v, deg_sh.at[dst_v0], add=True)

        @pl.when(j < NUM_CHUNKS // 2 - 1)
        def _prefetch():
            load_idx(c0 + 2, src_v0, dst_v0)
            pltpu.async_copy(p_hbm.at[src_v0], rows0, sem0)

        pltpu.make_async_copy(p_hbm.at[src_v1], rows1, sem1).wait()
        pltpu.sync_copy(rows1, acc_sh.at[dst_v1], add=True)
        if compute_deg:
            pltpu.sync_copy(ones_v, deg_sh.at[dst_v1], add=True)

    plsc.subcore_barrier()
    pltpu.sync_copy(acc_sh.at[pl.ds(r0, ROWS_PER_TILE)],
                    acc_out.at[cid, pl.ds(r0, ROWS_PER_TILE)])
    if compute_deg:
        pltpu.sync_copy(deg_sh.at[pl.ds(r0, ROWS_PER_TILE)],
                        deg_out.at[cid, pl.ds(r0, ROWS_PER_TILE)])


def _make_sc_kernel(compute_deg):
    out_type = [jax.ShapeDtypeStruct((NC, N_PAD, F), jnp.float32)]
    scratch = [
        pltpu.VMEM((CHUNK,), jnp.int32),
        pltpu.VMEM((CHUNK,), jnp.int32),
        pltpu.VMEM((CHUNK,), jnp.int32),
        pltpu.VMEM((CHUNK,), jnp.int32),
        pltpu.VMEM((CHUNK, F), jnp.float32),
        pltpu.VMEM((CHUNK, F), jnp.float32),
    ]
    if compute_deg:
        out_type.append(jax.ShapeDtypeStruct((NC, N_PAD), jnp.float32))
        scratch.append(pltpu.VMEM((CHUNK,), jnp.float32))
    scratch.append(pltpu.VMEM_SHARED((N_PAD, F), jnp.float32))
    if compute_deg:
        scratch.append(pltpu.VMEM_SHARED((N_PAD,), jnp.float32))
    scratch.append(pltpu.SemaphoreType.DMA)
    scratch.append(pltpu.SemaphoreType.DMA)
    return pl.kernel(
        functools.partial(_sc_body, compute_deg),
        out_type=tuple(out_type),
        mesh=plsc.VectorSubcoreMesh(core_axis_name="c", subcore_axis_name="s"),
        scratch_types=tuple(scratch),
    )


def _mm_in_body(x_ref, w_ref, b_ref, p_ref, r_ref):
    o = lax.dot_general(x_ref[...], w_ref[...], (((1,), (1,)), ((), ())),
                        preferred_element_type=jnp.float32)
    p_ref[...] = o[:, :F]
    r_ref[...] = o[:, F:] + b_ref[...]


def _fuse_body(parts_ref, degp_ref, r_ref, g_ref, be_ref, w_ref, b_ref,
               p_ref, rout_ref):
    acc = parts_ref[0] + parts_ref[1]
    deg = jnp.maximum(jnp.sum(degp_ref[...], axis=0), 1.0)
    h = acc / deg[:, None] + r_ref[...]
    h = jnp.maximum(g_ref[...] * (h * C_BN) + be_ref[...], 0.0)
    o = lax.dot_general(h, w_ref[...], (((1,), (1,)), ((), ())),
                        preferred_element_type=jnp.float32)
    p_ref[...] = o[:, :F]
    rout_ref[...] = o[:, F:] + b_ref[...]


def _head_body(parts_ref, degp_ref, r_ref, g_ref, be_ref, w_ref, b_ref,
               out_ref):
    acc = parts_ref[0] + parts_ref[1]
    deg = jnp.maximum(jnp.sum(degp_ref[...], axis=0), 1.0)
    h = acc / deg[:, None] + r_ref[...]
    h = jnp.maximum(g_ref[...] * (h * C_BN) + be_ref[...], 0.0)
    out_ref[...] = lax.dot_general(h, w_ref[...], (((1,), (1,)), ((), ())),
                                   preferred_element_type=jnp.float32) + b_ref[...]


_GRID = N_PAD // BLK
_row = pl.BlockSpec((BLK, F), lambda i: (i, 0))
_full2 = lambda a, b: pl.BlockSpec((a, b), lambda i: (0, 0))
_parts_spec = pl.BlockSpec((NC, BLK, F), lambda i: (0, i, 0))
_degp_spec = pl.BlockSpec((NC, BLK), lambda i: (0, i))

_mm_in = pl.pallas_call(
    _mm_in_body,
    grid=(_GRID,),
    in_specs=[_row, _full2(2 * F, F), _full2(1, F)],
    out_specs=[_row, _row],
    out_shape=[jax.ShapeDtypeStruct((N_PAD, F), jnp.float32)] * 2,
)

_fuse = pl.pallas_call(
    _fuse_body,
    grid=(_GRID,),
    in_specs=[_parts_spec, _degp_spec, _row, _full2(1, F), _full2(1, F),
              _full2(2 * F, F), _full2(1, F)],
    out_specs=[_row, _row],
    out_shape=[jax.ShapeDtypeStruct((N_PAD, F), jnp.float32)] * 2,
)

_head = pl.pallas_call(
    _head_body,
    grid=(_GRID,),
    in_specs=[_parts_spec, _degp_spec, _row, _full2(1, F), _full2(1, F),
              _full2(F, F), _full2(1, F)],
    out_specs=_row,
    out_shape=jax.ShapeDtypeStruct((N_PAD, F), jnp.float32),
)


def kernel(x, ei, W1l, b1l, W1r, W2l, b2l, W2r, W3l, b3l, W3r,
           g1, be1, g2, be2, g3, be3, Wh, bh):
    x_pad = jnp.pad(x, ((0, N_PAD - N), (0, 0)))
    pad_i = jnp.arange(E_PAD - E, dtype=jnp.int32)
    src = jnp.concatenate([ei[0], pad_i * 13 % N])
    dst = jnp.concatenate([ei[1], N + pad_i % (N_PAD - N)])
    zeros = jnp.zeros((N_PAD, F), jnp.float32)
    zeros1 = jnp.zeros((N_PAD,), jnp.float32)

    row = lambda v: v.reshape(1, F)
    Wc1 = jnp.concatenate([W1l, W1r], axis=0)
    Wc2 = jnp.concatenate([W2l, W2r], axis=0)
    Wc3 = jnp.concatenate([W3l, W3r], axis=0)

    agg_deg = _make_sc_kernel(True)
    agg = _make_sc_kernel(False)

    p1, r1 = _mm_in(x_pad, Wc1, row(b1l))
    parts1, degp = agg_deg(p1, src, dst, zeros, zeros1)
    p2, r2 = _fuse(parts1, degp, r1, row(g1), row(be1), Wc2, row(b2l))
    (parts2,) = agg(p2, src, dst, zeros)
    p3, r3 = _fuse(parts2, degp, r2, row(g2), row(be2), Wc3, row(b3l))
    (parts3,) = agg(p3, src, dst, zeros)
    out = _head(parts3, degp, r3, row(g3), row(be3), Wh, row(bh))
    return out[:N]
